# Initial kernel scaffold; baseline (speedup 1.0000x reference)
import jax
import jax.numpy as jnp
from jax.experimental import pallas as pl


def kernel(x_nchw, w, b, gamma, beta):
    raise NotImplementedError("write your pallas kernel here")


# speedup vs baseline: 3.2434x; 3.2434x over previous
import functools

import jax
import jax.numpy as jnp
from jax.experimental import pallas as pl
from jax.experimental.pallas import tpu as pltpu


def _conv_pool_stats_kernel(a_ref, b_ref, w_ref, out_ref, stats_ref, *, TH, W, C):
    THp, Wp = TH // 2, W // 2
    x = jnp.concatenate([a_ref[0], b_ref[0]], axis=0)
    Cin = x.shape[-1]

    xc = jnp.concatenate(
        [x[:, 0:W, :], x[:, 1:W + 1, :], x[:, 2:W + 2, :]], axis=-1)

    a = jnp.dot(xc.reshape((TH + 2) * W, 3 * Cin), w_ref[...],
                preferred_element_type=jnp.float32)
    a = a.reshape(TH + 2, W, 3 * C)

    conv = (a[0:TH, :, 0:C]
            + a[1:TH + 1, :, C:2 * C]
            + a[2:TH + 2, :, 2 * C:3 * C])

    c = conv.reshape(THp, 2, W, C)
    mh = jnp.maximum(c[:, 0], c[:, 1])
    mh2 = mh.reshape(THp, Wp, 2 * C)
    pooled = jnp.maximum(mh2[:, :, :C], mh2[:, :, C:])
    pooled = pooled.reshape(THp * Wp, C)
    out_ref[0] = pooled

    s = jnp.sum(pooled, axis=0, keepdims=True)
    ss = jnp.sum(pooled * pooled, axis=0, keepdims=True)
    stats_ref[0] = jnp.concatenate([s, ss], axis=0)


def _bn_relu_kernel(x_ref, scale_ref, shift_ref, o_ref):
    o_ref[...] = jnp.maximum(x_ref[...] * scale_ref[...] + shift_ref[...], 0.0)


def kernel(x_nchw, w, b, gamma, beta):
    del b
    eps = 1e-5
    N, Cin, H, W = x_nchw.shape
    Cout = w.shape[0]
    assert H % 2 == 0 and W % 2 == 0
    Hp, Wp = H // 2, W // 2

    TH = 32
    while H % TH != 0:
        TH //= 2
    THp = TH // 2
    nH = H // TH

    x = jnp.transpose(x_nchw, (0, 2, 3, 1)).astype(jnp.bfloat16)
    xp = jnp.pad(x, ((0, 0), (1, 1), (1, 1), (0, 0)))
    wN = jnp.transpose(w, (3, 1, 2, 0)).reshape(3 * Cin, 3 * Cout)
    wN = wN.astype(jnp.bfloat16)

    cparams = pltpu.CompilerParams(
        dimension_semantics=("parallel", "parallel"),
        vmem_limit_bytes=100 * 1024 * 1024,
    )

    k1 = functools.partial(_conv_pool_stats_kernel, TH=TH, W=W, C=Cout)
    pooled, stats = pl.pallas_call(
        k1,
        grid=(N, nH),
        in_specs=[
            pl.BlockSpec((1, TH, W + 2, Cin), lambda n, h: (n, h, 0, 0)),
            pl.BlockSpec((1, 2, W + 2, Cin), lambda n, h: (n, THp * (h + 1), 0, 0)),
            pl.BlockSpec((3 * Cin, 3 * Cout), lambda n, h: (0, 0)),
        ],
        out_specs=[
            pl.BlockSpec((1, THp * Wp, Cout), lambda n, h: (n, h, 0)),
            pl.BlockSpec((1, 2, Cout), lambda n, h: (n * nH + h, 0, 0)),
        ],
        out_shape=[
            jax.ShapeDtypeStruct((N, Hp * Wp, Cout), jnp.float32),
            jax.ShapeDtypeStruct((N * nH, 2, Cout), jnp.float32),
        ],
        compiler_params=cparams,
    )(xp, xp, wN)

    M2 = N * Hp * Wp
    ssum = jnp.sum(stats, axis=0)
    mean = ssum[0] / M2
    var = jnp.maximum(ssum[1] / M2 - mean * mean, 0.0)
    scale = gamma.astype(jnp.float32) * jax.lax.rsqrt(var + eps)
    shift = beta.astype(jnp.float32) - mean * scale
    scale2 = scale.reshape(1, Cout)
    shift2 = shift.reshape(1, Cout)

    TR = 4096
    while M2 % TR != 0:
        TR //= 2
    pooled2d = pooled.reshape(M2, Cout)
    out2d = pl.pallas_call(
        _bn_relu_kernel,
        grid=(M2 // TR,),
        in_specs=[
            pl.BlockSpec((TR, Cout), lambda i: (i, 0)),
            pl.BlockSpec((1, Cout), lambda i: (0, 0)),
            pl.BlockSpec((1, Cout), lambda i: (0, 0)),
        ],
        out_specs=pl.BlockSpec((TR, Cout), lambda i: (i, 0)),
        out_shape=jax.ShapeDtypeStruct((M2, Cout), jnp.float32),
        compiler_params=pltpu.CompilerParams(
            dimension_semantics=("parallel",),
            vmem_limit_bytes=64 * 1024 * 1024,
        ),
    )(pooled2d, scale2, shift2)

    out = out2d.reshape(N, Hp, Wp, Cout)
    return jnp.transpose(out, (0, 3, 1, 2))
